# baseline (speedup 1.0000x reference)
import functools
import numpy as np
import jax
import jax.numpy as jnp
from jax import lax
from jax.experimental import pallas as pl
from jax.experimental.pallas import tpu as pltpu
from jax.experimental.pallas import tpu_sc as plsc

N = 10000
E = 320000
IN_DIM = 128
H0 = 8
OD = 8
NCLS = 16
F1 = 80
F2 = 32

NCORES = 2
NSUB = 16
NW = NCORES * NSUB
CHUNK = 96
NBUF = 3
ISLOTS = 12
ETOT = 2 * E
CHUNKS = ISLOTS * (-(-ETOT // (NW * CHUNK * ISLOTS)))
EPT = CHUNKS * CHUNK
EPAD = EPT * NW
ACC = 20096
RPT = ACC // NSUB
DUMMY = 2 * N
EPS = 1e-30


def _sel(rows, cols):
    i = lax.broadcasted_iota(jnp.int32, (rows, cols), 0)
    j = lax.broadcasted_iota(jnp.int32, (rows, cols), 1)
    return (i == j).astype(jnp.float32)


def _headpool(hd, h):
    d = hd // h
    i = lax.broadcasted_iota(jnp.int32, (hd, h), 0)
    j = lax.broadcasted_iota(jnp.int32, (hd, h), 1)
    return (i // d == j).astype(jnp.float32)


def _den_expand(f, hd, h):
    d = hd // h
    i = lax.broadcasted_iota(jnp.int32, (f, hd), 0)
    j = lax.broadcasted_iota(jnp.int32, (f, hd), 1)
    return (i == hd + j // d).astype(jnp.float32)


def _prep1_body(x_ref, w_ref, al_ref, g_ref):
    h = jnp.dot(x_ref[...], w_ref[...], preferred_element_type=jnp.float32)
    t = h * al_ref[...]
    m = _headpool(H0 * OD, H0)
    el = jnp.dot(t, m, preferred_element_type=jnp.float32)
    wgt = jnp.exp(el)
    we = jnp.dot(wgt, m.T, preferred_element_type=jnp.float32)
    g_ref[...] = jnp.concatenate([h * we, wgt, jnp.zeros_like(wgt)], axis=1)


def _fin1_prep2_body(p_ref, b1_ref, w2_ref, al2_ref, g2_ref):
    a1 = p_ref[0:N, :] + p_ref[ACC:ACC + N, :]
    a2 = p_ref[N:2 * N, :] + p_ref[ACC + N:ACC + 2 * N, :]
    msel = _sel(F1, H0 * OD)
    mden = _den_expand(F1, H0 * OD, H0)
    x1 = (jnp.dot(a1, msel, preferred_element_type=jnp.float32)
          / jnp.maximum(jnp.dot(a1, mden, preferred_element_type=jnp.float32), EPS)
          + jnp.dot(a2, msel, preferred_element_type=jnp.float32)
          / jnp.maximum(jnp.dot(a2, mden, preferred_element_type=jnp.float32), EPS)
          + 2.0 * b1_ref[...])
    h2 = jnp.dot(x1, w2_ref[...], preferred_element_type=jnp.float32)
    t2 = h2 * al2_ref[...]
    el2 = jnp.dot(t2, jnp.ones((NCLS, 1), jnp.float32),
                  preferred_element_type=jnp.float32)
    w2 = jnp.exp(el2)
    g2_ref[...] = jnp.concatenate(
        [h2 * w2, w2, jnp.zeros((N, F2 - NCLS - 1), jnp.float32)], axis=1)


def _fin2_body(p_ref, b2_ref, out_ref):
    a1 = p_ref[0:N, :] + p_ref[ACC:ACC + N, :]
    a2 = p_ref[N:2 * N, :] + p_ref[ACC + N:ACC + 2 * N, :]
    msel = _sel(F2, NCLS)
    mden = _den_expand(F2, NCLS, 1)
    out_ref[...] = (jnp.dot(a1, msel, preferred_element_type=jnp.float32)
                    / jnp.maximum(jnp.dot(a1, mden, preferred_element_type=jnp.float32), EPS)
                    + jnp.dot(a2, msel, preferred_element_type=jnp.float32)
                    / jnp.maximum(jnp.dot(a2, mden, preferred_element_type=jnp.float32), EPS)
                    + 2.0 * b2_ref[...])


_prep1 = pl.pallas_call(
    _prep1_body,
    out_shape=jax.ShapeDtypeStruct((N, F1), jnp.float32),
)

_fin1_prep2 = pl.pallas_call(
    _fin1_prep2_body,
    out_shape=jax.ShapeDtypeStruct((N, F2), jnp.float32),
)

_fin2 = pl.pallas_call(
    _fin2_body,
    out_shape=jax.ShapeDtypeStruct((N, NCLS), jnp.float32),
)


def _make_scatter(F):
    mesh = plsc.VectorSubcoreMesh(
        core_axis_name="c", subcore_axis_name="s",
        num_cores=NCORES, num_subcores=NSUB)

    @functools.partial(
        pl.kernel,
        out_type=jax.ShapeDtypeStruct((2 * ACC, F), jnp.float32),
        mesh=mesh,
        scratch_types=[
            [pltpu.VMEM((CHUNK,), jnp.int32) for _ in range(ISLOTS)],
            [pltpu.VMEM((CHUNK,), jnp.int32) for _ in range(ISLOTS)],
            [pltpu.VMEM((CHUNK, F), jnp.float32) for _ in range(NBUF)],
            pltpu.VMEM_SHARED((ACC, F), jnp.float32),
            [pltpu.SemaphoreType.DMA for _ in range(NBUF)],
            [pltpu.SemaphoreType.DMA for _ in range(ISLOTS)],
            [pltpu.SemaphoreType.DMA for _ in range(NBUF)],
        ],
        compiler_params=pltpu.CompilerParams(use_tc_tiling_on_sc=False),
    )
    def scatter(g_hbm, src_hbm, dst_hbm, zeros_hbm, out_hbm,
                sidx, didx, rows, accum, gsem, isem, ssem):
        c = lax.axis_index("c")
        s = lax.axis_index("s")
        wid = s * NCORES + c
        stripe = s * RPT
        base_e = wid * EPT

        def fetch_idx(i, slot):
            pltpu.async_copy(src_hbm.at[pl.ds(base_e + i * CHUNK, CHUNK)],
                             sidx[slot], isem[slot])
            pltpu.async_copy(dst_hbm.at[pl.ds(base_e + i * CHUNK, CHUNK)],
                             didx[slot], isem[slot])

        def wait_idx(slot):
            pltpu.make_async_copy(src_hbm.at[pl.ds(0, CHUNK)],
                                  sidx[slot], isem[slot]).wait()
            pltpu.make_async_copy(src_hbm.at[pl.ds(0, CHUNK)],
                                  didx[slot], isem[slot]).wait()

        def start_gather(slot, b):
            pltpu.async_copy(g_hbm.at[sidx[slot]], rows[b], gsem[b])

        def wait_gather(b):
            pltpu.make_async_copy(g_hbm.at[sidx[0]], rows[b], gsem[b]).wait()

        def start_scatter(u, b):
            pltpu.async_copy(rows[b], accum.at[didx[u]], ssem[b], add=True)

        def wait_scatter(b):
            pltpu.make_async_copy(rows[b], accum.at[didx[0]], ssem[b]).wait()

        for u in range(ISLOTS // 2):
            fetch_idx(u, u)
        pltpu.sync_copy(zeros_hbm, accum.at[pl.ds(stripe, RPT)])
        for b in range(2):
            wait_idx(b)
            start_gather(b, b)
        plsc.subcore_barrier()

        def body(q, carry):
            for u in range(ISLOTS):
                i = ISLOTS * q + u
                b = u % NBUF
                wait_gather(b)
                start_scatter(u, b)

                @pl.when((i + 2 < CHUNKS) & (i >= 1))
                def _():
                    wait_scatter((u + 2) % NBUF)

                @pl.when(i + 2 < CHUNKS)
                def _():
                    wait_idx((u + 2) % ISLOTS)
                    start_gather((u + 2) % ISLOTS, (u + 2) % NBUF)

                @pl.when(i + ISLOTS // 2 < CHUNKS)
                def _():
                    fetch_idx(i + ISLOTS // 2, (u + ISLOTS // 2) % ISLOTS)
            return carry

        lax.fori_loop(0, CHUNKS // ISLOTS, body, 0)
        for b in range(NBUF):
            wait_scatter(b)
        plsc.subcore_barrier()
        pltpu.sync_copy(accum.at[pl.ds(stripe, RPT)],
                        out_hbm.at[pl.ds(c * ACC + stripe, RPT)])

    return scatter


_scatter1 = _make_scatter(F1)
_scatter2 = _make_scatter(F2)


def kernel(x, edge_index1, edge_index2, W1, al1, ar1, b1, W2, al2, ar2, b2):
    pad = EPAD - ETOT
    src = jnp.concatenate([edge_index1[0], edge_index2[0],
                           jnp.zeros((pad,), jnp.int32)])
    dst = jnp.concatenate([edge_index1[1], edge_index2[1] + N,
                           jnp.full((pad,), DUMMY, jnp.int32)])
    zeros1 = jnp.zeros((RPT, F1), jnp.float32)
    zeros2 = jnp.zeros((RPT, F2), jnp.float32)

    g1 = _prep1(x, W1, al1.reshape(1, H0 * OD))
    p1 = _scatter1(g1, src, dst, zeros1)
    g2 = _fin1_prep2(p1, b1.reshape(1, H0 * OD), W2, al2.reshape(1, NCLS))
    p2 = _scatter2(g2, src, dst, zeros2)
    return _fin2(p2, b2.reshape(1, NCLS))


# speedup vs baseline: 3.2663x; 3.2663x over previous
import functools
import numpy as np
import jax
import jax.numpy as jnp
from jax import lax
from jax.experimental import pallas as pl
from jax.experimental.pallas import tpu as pltpu
from jax.experimental.pallas import tpu_sc as plsc

N = 10000
E = 320000
IN_DIM = 128
H0 = 8
OD = 8
NCLS = 16
F1 = 80
F2 = 32

NCORES = 2
NSUB = 16
NW = NCORES * NSUB
CHUNK = 128
NBUF = 2
ISLOTS = 4
ETOT = 2 * E
CHUNKS = ISLOTS * (-(-ETOT // (NW * CHUNK * ISLOTS)))
EPT = CHUNKS * CHUNK
EPAD = EPT * NW
ACC = 21120
RPT = ACC // NSUB
DUMMY = 2 * N
EPS = 1e-30


def _sel(rows, cols):
    i = lax.broadcasted_iota(jnp.int32, (rows, cols), 0)
    j = lax.broadcasted_iota(jnp.int32, (rows, cols), 1)
    return (i == j).astype(jnp.float32)


def _headpool(hd, h):
    d = hd // h
    i = lax.broadcasted_iota(jnp.int32, (hd, h), 0)
    j = lax.broadcasted_iota(jnp.int32, (hd, h), 1)
    return (i // d == j).astype(jnp.float32)


def _den_expand(f, hd, h):
    d = hd // h
    i = lax.broadcasted_iota(jnp.int32, (f, hd), 0)
    j = lax.broadcasted_iota(jnp.int32, (f, hd), 1)
    return (i == hd + j // d).astype(jnp.float32)


def _prep1_body(x_ref, w_ref, al_ref, g_ref):
    h = jnp.dot(x_ref[...], w_ref[...], preferred_element_type=jnp.float32)
    t = h * al_ref[...]
    m = _headpool(H0 * OD, H0)
    el = jnp.dot(t, m, preferred_element_type=jnp.float32)
    wgt = jnp.exp(el)
    we = jnp.dot(wgt, m.T, preferred_element_type=jnp.float32)
    g_ref[...] = jnp.concatenate([h * we, wgt, jnp.zeros_like(wgt)], axis=1)


def _fin1_prep2_body(p_ref, b1_ref, w2_ref, al2_ref, g2_ref):
    a1 = p_ref[0:N, :] + p_ref[ACC:ACC + N, :]
    a2 = p_ref[N:2 * N, :] + p_ref[ACC + N:ACC + 2 * N, :]
    msel = _sel(F1, H0 * OD)
    mden = _den_expand(F1, H0 * OD, H0)
    x1 = (jnp.dot(a1, msel, preferred_element_type=jnp.float32)
          / jnp.maximum(jnp.dot(a1, mden, preferred_element_type=jnp.float32), EPS)
          + jnp.dot(a2, msel, preferred_element_type=jnp.float32)
          / jnp.maximum(jnp.dot(a2, mden, preferred_element_type=jnp.float32), EPS)
          + 2.0 * b1_ref[...])
    h2 = jnp.dot(x1, w2_ref[...], preferred_element_type=jnp.float32)
    t2 = h2 * al2_ref[...]
    el2 = jnp.dot(t2, jnp.ones((NCLS, 1), jnp.float32),
                  preferred_element_type=jnp.float32)
    w2 = jnp.exp(el2)
    g2_ref[...] = jnp.concatenate(
        [h2 * w2, w2, jnp.zeros((N, F2 - NCLS - 1), jnp.float32)], axis=1)


def _fin2_body(p_ref, b2_ref, out_ref):
    a1 = p_ref[0:N, :] + p_ref[ACC:ACC + N, :]
    a2 = p_ref[N:2 * N, :] + p_ref[ACC + N:ACC + 2 * N, :]
    msel = _sel(F2, NCLS)
    mden = _den_expand(F2, NCLS, 1)
    out_ref[...] = (jnp.dot(a1, msel, preferred_element_type=jnp.float32)
                    / jnp.maximum(jnp.dot(a1, mden, preferred_element_type=jnp.float32), EPS)
                    + jnp.dot(a2, msel, preferred_element_type=jnp.float32)
                    / jnp.maximum(jnp.dot(a2, mden, preferred_element_type=jnp.float32), EPS)
                    + 2.0 * b2_ref[...])


_prep1 = pl.pallas_call(
    _prep1_body,
    out_shape=jax.ShapeDtypeStruct((N, F1), jnp.float32),
)

_fin1_prep2 = pl.pallas_call(
    _fin1_prep2_body,
    out_shape=jax.ShapeDtypeStruct((N, F2), jnp.float32),
)

_fin2 = pl.pallas_call(
    _fin2_body,
    out_shape=jax.ShapeDtypeStruct((N, NCLS), jnp.float32),
)


def _make_scatter(F):
    mesh = plsc.VectorSubcoreMesh(
        core_axis_name="c", subcore_axis_name="s",
        num_cores=NCORES, num_subcores=NSUB)

    @functools.partial(
        pl.kernel,
        out_type=jax.ShapeDtypeStruct((2 * ACC, F), jnp.float32),
        mesh=mesh,
        scratch_types=[
            [pltpu.VMEM((CHUNK,), jnp.int32) for _ in range(ISLOTS)],
            [pltpu.VMEM((CHUNK,), jnp.int32) for _ in range(ISLOTS)],
            [pltpu.VMEM((CHUNK, F), jnp.float32) for _ in range(NBUF)],
            pltpu.VMEM_SHARED((ACC, F), jnp.float32),
            [pltpu.SemaphoreType.DMA for _ in range(NBUF)],
            [pltpu.SemaphoreType.DMA for _ in range(ISLOTS)],
        ],
        compiler_params=pltpu.CompilerParams(use_tc_tiling_on_sc=False),
    )
    def scatter(g_hbm, src_hbm, dst_hbm, zeros_hbm, out_hbm,
                sidx, didx, rows, accum, gsem, isem):
        c = lax.axis_index("c")
        s = lax.axis_index("s")
        wid = s * NCORES + c
        stripe = s * RPT
        base_e = wid * EPT

        def fetch_idx(i, slot):
            pltpu.async_copy(src_hbm.at[pl.ds(base_e + i * CHUNK, CHUNK)],
                             sidx[slot], isem[slot])
            pltpu.async_copy(dst_hbm.at[pl.ds(base_e + i * CHUNK, CHUNK)],
                             didx[slot], isem[slot])

        def wait_idx(slot):
            pltpu.make_async_copy(src_hbm.at[pl.ds(0, CHUNK)],
                                  sidx[slot], isem[slot]).wait()
            pltpu.make_async_copy(src_hbm.at[pl.ds(0, CHUNK)],
                                  didx[slot], isem[slot]).wait()

        def start_gather(slot, b):
            pltpu.async_copy(g_hbm.at[sidx[slot]], rows[b], gsem[b])

        def wait_gather(b):
            pltpu.make_async_copy(g_hbm.at[sidx[0]], rows[b], gsem[b]).wait()

        for u in range(ISLOTS):
            fetch_idx(u, u)
        pltpu.sync_copy(zeros_hbm, accum.at[pl.ds(stripe, RPT)])
        for b in range(NBUF):
            wait_idx(b)
            start_gather(b, b)
        plsc.subcore_barrier()

        def body(q, carry):
            for u in range(ISLOTS):
                i = ISLOTS * q + u
                b = u % NBUF
                wait_gather(b)
                pltpu.sync_copy(rows[b], accum.at[didx[u]], add=True)

                @pl.when(i + NBUF < CHUNKS)
                def _():
                    wait_idx((u + NBUF) % ISLOTS)
                    start_gather((u + NBUF) % ISLOTS, b)

                @pl.when(i + ISLOTS < CHUNKS)
                def _():
                    fetch_idx(i + ISLOTS, u)
            return carry

        lax.fori_loop(0, CHUNKS // ISLOTS, body, 0)
        plsc.subcore_barrier()
        pltpu.sync_copy(accum.at[pl.ds(stripe, RPT)],
                        out_hbm.at[pl.ds(c * ACC + stripe, RPT)])

    return scatter


_scatter1 = _make_scatter(F1)
_scatter2 = _make_scatter(F2)


def kernel(x, edge_index1, edge_index2, W1, al1, ar1, b1, W2, al2, ar2, b2):
    pad = EPAD - ETOT
    pad_iota = jnp.arange(pad, dtype=jnp.int32)
    src = jnp.concatenate([edge_index1[0], edge_index2[0], pad_iota % N])
    dst = jnp.concatenate([edge_index1[1], edge_index2[1] + N,
                           DUMMY + pad_iota % (ACC - DUMMY)])
    zeros1 = jnp.zeros((RPT, F1), jnp.float32)
    zeros2 = jnp.zeros((RPT, F2), jnp.float32)

    g1 = _prep1(x, W1, al1.reshape(1, H0 * OD))
    p1 = _scatter1(g1, src, dst, zeros1)
    g2 = _fin1_prep2(p1, b1.reshape(1, H0 * OD), W2, al2.reshape(1, NCLS))
    p2 = _scatter2(g2, src, dst, zeros2)
    return _fin2(p2, b2.reshape(1, NCLS))


# speedup vs baseline: 3.4550x; 1.0578x over previous
import functools
import numpy as np
import jax
import jax.numpy as jnp
from jax import lax
from jax.experimental import pallas as pl
from jax.experimental.pallas import tpu as pltpu
from jax.experimental.pallas import tpu_sc as plsc

N = 10000
E = 320000
IN_DIM = 128
H0 = 8
OD = 8
NCLS = 16
F1 = 80
F2 = 32

NCORES = 2
NSUB = 16
NW = NCORES * NSUB
CHUNK = 128
NBUF = 2
ISLOTS = 4
ETOT = 2 * E
CHUNKS = ISLOTS * (-(-ETOT // (NW * CHUNK * ISLOTS)))
EPT = CHUNKS * CHUNK
EPAD = EPT * NW
ACC = 21120
RPT = ACC // NSUB
DUMMY = 2 * N
EPS = 1e-30


def _sel(rows, cols):
    i = lax.broadcasted_iota(jnp.int32, (rows, cols), 0)
    j = lax.broadcasted_iota(jnp.int32, (rows, cols), 1)
    return (i == j).astype(jnp.float32)


def _headpool(hd, h):
    d = hd // h
    i = lax.broadcasted_iota(jnp.int32, (hd, h), 0)
    j = lax.broadcasted_iota(jnp.int32, (hd, h), 1)
    return (i // d == j).astype(jnp.float32)


def _den_expand(f, hd, h):
    d = hd // h
    i = lax.broadcasted_iota(jnp.int32, (f, hd), 0)
    j = lax.broadcasted_iota(jnp.int32, (f, hd), 1)
    return (i == hd + j // d).astype(jnp.float32)


def _prep1_body(x_ref, w_ref, al_ref, g_ref):
    h = jnp.dot(x_ref[...], w_ref[...], preferred_element_type=jnp.float32)
    t = h * al_ref[...]
    m = _headpool(H0 * OD, H0)
    el = jnp.dot(t, m, preferred_element_type=jnp.float32)
    wgt = jnp.exp(el)
    we = jnp.dot(wgt, m.T, preferred_element_type=jnp.float32)
    g_ref[...] = jnp.concatenate([h * we, wgt, jnp.zeros_like(wgt)], axis=1)


def _fin1_prep2_body(p_ref, b1_ref, w2_ref, al2_ref, g2_ref):
    a1 = p_ref[0:N, :] + p_ref[ACC:ACC + N, :]
    a2 = p_ref[N:2 * N, :] + p_ref[ACC + N:ACC + 2 * N, :]
    msel = _sel(F1, H0 * OD)
    mden = _den_expand(F1, H0 * OD, H0)
    x1 = (jnp.dot(a1, msel, preferred_element_type=jnp.float32)
          / jnp.maximum(jnp.dot(a1, mden, preferred_element_type=jnp.float32), EPS)
          + jnp.dot(a2, msel, preferred_element_type=jnp.float32)
          / jnp.maximum(jnp.dot(a2, mden, preferred_element_type=jnp.float32), EPS)
          + 2.0 * b1_ref[...])
    h2 = jnp.dot(x1, w2_ref[...], preferred_element_type=jnp.float32)
    t2 = h2 * al2_ref[...]
    el2 = jnp.dot(t2, jnp.ones((NCLS, 1), jnp.float32),
                  preferred_element_type=jnp.float32)
    w2 = jnp.exp(el2)
    g2_ref[...] = jnp.concatenate(
        [h2 * w2, w2, jnp.zeros((N, F2 - NCLS - 1), jnp.float32)], axis=1)


def _fin2_body(p_ref, b2_ref, out_ref):
    a1 = p_ref[0:N, :] + p_ref[ACC:ACC + N, :]
    a2 = p_ref[N:2 * N, :] + p_ref[ACC + N:ACC + 2 * N, :]
    msel = _sel(F2, NCLS)
    mden = _den_expand(F2, NCLS, 1)
    out_ref[...] = (jnp.dot(a1, msel, preferred_element_type=jnp.float32)
                    / jnp.maximum(jnp.dot(a1, mden, preferred_element_type=jnp.float32), EPS)
                    + jnp.dot(a2, msel, preferred_element_type=jnp.float32)
                    / jnp.maximum(jnp.dot(a2, mden, preferred_element_type=jnp.float32), EPS)
                    + 2.0 * b2_ref[...])


_prep1 = pl.pallas_call(
    _prep1_body,
    out_shape=jax.ShapeDtypeStruct((N, F1), jnp.float32),
)

_fin1_prep2 = pl.pallas_call(
    _fin1_prep2_body,
    out_shape=jax.ShapeDtypeStruct((N, F2), jnp.float32),
)

_fin2 = pl.pallas_call(
    _fin2_body,
    out_shape=jax.ShapeDtypeStruct((N, NCLS), jnp.float32),
)


def _make_scatter(F):
    mesh = plsc.VectorSubcoreMesh(
        core_axis_name="c", subcore_axis_name="s",
        num_cores=NCORES, num_subcores=NSUB)

    @functools.partial(
        pl.kernel,
        out_type=jax.ShapeDtypeStruct((2 * ACC, F), jnp.float32),
        mesh=mesh,
        scratch_types=[
            [pltpu.VMEM((CHUNK,), jnp.int32) for _ in range(ISLOTS)],
            [pltpu.VMEM((CHUNK,), jnp.int32) for _ in range(ISLOTS)],
            [pltpu.VMEM((CHUNK, F), jnp.float32) for _ in range(NBUF)],
            pltpu.VMEM_SHARED((ACC, F), jnp.float32),
            [pltpu.SemaphoreType.DMA for _ in range(NBUF)],
            [pltpu.SemaphoreType.DMA for _ in range(ISLOTS)],
        ],
        compiler_params=pltpu.CompilerParams(use_tc_tiling_on_sc=False),
    )
    def scatter(g_hbm, src_hbm, dst_hbm, zeros_hbm, out_hbm,
                sidx, didx, rows, accum, gsem, isem):
        c = lax.axis_index("c")
        s = lax.axis_index("s")
        wid = s * NCORES + c
        stripe = s * RPT
        base_e = wid * EPT

        def fetch_idx(i, slot):
            pltpu.async_copy(src_hbm.at[pl.ds(base_e + i * CHUNK, CHUNK)],
                             sidx[slot], isem[slot])
            pltpu.async_copy(dst_hbm.at[pl.ds(base_e + i * CHUNK, CHUNK)],
                             didx[slot], isem[slot])

        def wait_idx(slot):
            pltpu.make_async_copy(src_hbm.at[pl.ds(0, CHUNK)],
                                  sidx[slot], isem[slot]).wait()
            pltpu.make_async_copy(src_hbm.at[pl.ds(0, CHUNK)],
                                  didx[slot], isem[slot]).wait()

        def start_gather(slot, b):
            pltpu.async_copy(g_hbm.at[sidx[slot]], rows[b], gsem[b])

        def wait_gather(b):
            pltpu.make_async_copy(g_hbm.at[sidx[0]], rows[b], gsem[b]).wait()

        for u in range(ISLOTS):
            fetch_idx(u, u)
        pltpu.sync_copy(zeros_hbm, accum.at[pl.ds(stripe, RPT)])
        for b in range(NBUF):
            wait_idx(b)
            start_gather(b, b)
        plsc.subcore_barrier()

        def body(q, carry):
            for u in range(ISLOTS):
                i = ISLOTS * q + u
                b = u % NBUF
                wait_gather(b)
                pltpu.sync_copy(rows[b], accum.at[didx[u]], add=True)

                @pl.when(i + NBUF < CHUNKS)
                def _():
                    wait_idx((u + NBUF) % ISLOTS)
                    start_gather((u + NBUF) % ISLOTS, b)

                @pl.when(i + ISLOTS < CHUNKS)
                def _():
                    fetch_idx(i + ISLOTS, u)
            return carry

        lax.fori_loop(0, CHUNKS // ISLOTS, body, 0)
        plsc.subcore_barrier()
        pltpu.sync_copy(accum.at[pl.ds(stripe, RPT)],
                        out_hbm.at[pl.ds(c * ACC + stripe, RPT)])

    return scatter


def _make_scatter_staged(F, NB, IS):
    assert CHUNKS % IS == 0 and IS % NB == 0
    GROWS = N // NSUB
    mesh = plsc.VectorSubcoreMesh(
        core_axis_name="c", subcore_axis_name="s",
        num_cores=NCORES, num_subcores=NSUB)

    @functools.partial(
        pl.kernel,
        out_type=jax.ShapeDtypeStruct((2 * ACC, F), jnp.float32),
        mesh=mesh,
        scratch_types=[
            [pltpu.VMEM((CHUNK,), jnp.int32) for _ in range(IS)],
            [pltpu.VMEM((CHUNK,), jnp.int32) for _ in range(IS)],
            [pltpu.VMEM((CHUNK, F), jnp.float32) for _ in range(NB)],
            pltpu.VMEM_SHARED((N, F), jnp.float32),
            pltpu.VMEM_SHARED((ACC, F), jnp.float32),
            [pltpu.SemaphoreType.DMA for _ in range(NB)],
            [pltpu.SemaphoreType.DMA for _ in range(IS)],
        ],
        compiler_params=pltpu.CompilerParams(use_tc_tiling_on_sc=False),
    )
    def scatter(g_hbm, src_hbm, dst_hbm, zeros_hbm, out_hbm,
                sidx, didx, rows, g_sp, accum, gsem, isem):
        c = lax.axis_index("c")
        s = lax.axis_index("s")
        wid = s * NCORES + c
        stripe = s * RPT
        base_e = wid * EPT

        def fetch_idx(i, slot):
            pltpu.async_copy(src_hbm.at[pl.ds(base_e + i * CHUNK, CHUNK)],
                             sidx[slot], isem[slot])
            pltpu.async_copy(dst_hbm.at[pl.ds(base_e + i * CHUNK, CHUNK)],
                             didx[slot], isem[slot])

        def wait_idx(slot):
            pltpu.make_async_copy(src_hbm.at[pl.ds(0, CHUNK)],
                                  sidx[slot], isem[slot]).wait()
            pltpu.make_async_copy(src_hbm.at[pl.ds(0, CHUNK)],
                                  didx[slot], isem[slot]).wait()

        def start_gather(slot, b):
            pltpu.async_copy(g_sp.at[sidx[slot]], rows[b], gsem[b])

        def wait_gather(b):
            pltpu.make_async_copy(g_sp.at[sidx[0]], rows[b], gsem[b]).wait()

        for u in range(IS):
            fetch_idx(u, u)
        pltpu.sync_copy(g_hbm.at[pl.ds(s * GROWS, GROWS)],
                        g_sp.at[pl.ds(s * GROWS, GROWS)])
        pltpu.sync_copy(zeros_hbm, accum.at[pl.ds(stripe, RPT)])
        plsc.subcore_barrier()
        for b in range(NB):
            wait_idx(b)
            start_gather(b, b)

        def body(q, carry):
            for u in range(IS):
                i = IS * q + u
                b = u % NB
                wait_gather(b)
                pltpu.sync_copy(rows[b], accum.at[didx[u]], add=True)

                @pl.when(i + NB < CHUNKS)
                def _():
                    wait_idx((u + NB) % IS)
                    start_gather((u + NB) % IS, b)

                @pl.when(i + IS < CHUNKS)
                def _():
                    fetch_idx(i + IS, u)
            return carry

        lax.fori_loop(0, CHUNKS // IS, body, 0)
        plsc.subcore_barrier()
        pltpu.sync_copy(accum.at[pl.ds(stripe, RPT)],
                        out_hbm.at[pl.ds(c * ACC + stripe, RPT)])

    return scatter


_scatter1 = _make_scatter(F1)
_scatter2 = _make_scatter_staged(F2, 4, 8)


def kernel(x, edge_index1, edge_index2, W1, al1, ar1, b1, W2, al2, ar2, b2):
    pad = EPAD - ETOT
    pad_iota = jnp.arange(pad, dtype=jnp.int32)
    src = jnp.concatenate([edge_index1[0], edge_index2[0], pad_iota % N])
    dst = jnp.concatenate([edge_index1[1], edge_index2[1] + N,
                           DUMMY + pad_iota % (ACC - DUMMY)])
    zeros1 = jnp.zeros((RPT, F1), jnp.float32)
    zeros2 = jnp.zeros((RPT, F2), jnp.float32)

    g1 = _prep1(x, W1, al1.reshape(1, H0 * OD))
    p1 = _scatter1(g1, src, dst, zeros1)
    g2 = _fin1_prep2(p1, b1.reshape(1, H0 * OD), W2, al2.reshape(1, NCLS))
    p2 = _scatter2(g2, src, dst, zeros2)
    return _fin2(p2, b2.reshape(1, NCLS))
